# baseline (speedup 1.0000x reference)
import functools

import jax
import jax.numpy as jnp
from jax import lax
from jax.experimental import pallas as pl
from jax.experimental.pallas import tpu as pltpu
from jax.experimental.pallas import tpu_sc as plsc

N = 10000
E = 320000
D = 128
F = 384
NP = 10240
RT = NP // 16
CH = 128
NCH = 79
NFAST = 104
NSLOW = 56
NCHUNKS = 16 * (NFAST + NSLOW)
EP = NCHUNKS * CH
BR = 256

_MESH = plsc.VectorSubcoreMesh(core_axis_name="c", subcore_axis_name="s")


@functools.partial(
    pl.kernel,
    mesh=_MESH,
    out_type=jax.ShapeDtypeStruct((2, NP, 16), jnp.float32),
    scratch_types=[
        pltpu.VMEM((NCH, CH), jnp.int32),
        pltpu.VMEM((CH, 16), jnp.float32),
        pltpu.VMEM_SHARED((NP, 16), jnp.float32),
    ],
    compiler_params=pltpu.CompilerParams(use_tc_tiling_on_sc=False),
)
def _deg_kernel(dst_hbm, ones_hbm, zeros_hbm, out_hbm, dst_v, ones_v, acc):
    c = lax.axis_index("c")
    s = lax.axis_index("s")
    wid = c * 16 + s
    pltpu.sync_copy(zeros_hbm, acc.at[pl.ds(s * RT, RT)])
    pltpu.sync_copy(dst_hbm.at[wid], dst_v)
    pltpu.sync_copy(ones_hbm, ones_v)
    plsc.subcore_barrier()

    def body(j, carry):
        pltpu.sync_copy(ones_v, acc.at[dst_v.at[j]], add=True)
        return carry

    lax.fori_loop(0, NCH, body, 0)
    plsc.subcore_barrier()
    pltpu.sync_copy(acc.at[pl.ds(s * RT, RT)], out_hbm.at[c, pl.ds(s * RT, RT)])


@functools.partial(
    pl.kernel,
    mesh=_MESH,
    out_type=jax.ShapeDtypeStruct((3, 2, NP, D), jnp.float32),
    scratch_types=[
        pltpu.VMEM((2, CH), jnp.int32),
        pltpu.VMEM((NFAST, CH), jnp.int32),
        pltpu.VMEM((CH, D), jnp.float32),
        pltpu.VMEM((CH, D), jnp.float32),
        pltpu.SemaphoreType.DMA,
        pltpu.SemaphoreType.DMA,
        pltpu.SemaphoreType.DMA,
        pltpu.SemaphoreType.DMA,
        pltpu.VMEM_SHARED((NP, D), jnp.float32),
    ],
)
def _scatter_kernel(tab0, tab1, tab2, src_hbm, dst_hbm, zeros_hbm, out_hbm,
                    sidx, dst_v, bufa, bufb, semg0, semg1, semi0, semi1, acc):
    c = lax.axis_index("c")
    s = lax.axis_index("s")
    bufs = (bufa, bufb)
    semg = (semg0, semg1)
    semi = (semi0, semi1)

    def run(base, n):
        pltpu.sync_copy(dst_hbm.at[pl.ds(base, n)], dst_v.at[pl.ds(0, n)])

        def start_sidx(j, b):
            pltpu.async_copy(src_hbm.at[base + j], sidx.at[b], semi[b])

        def wait_sidx(j, b):
            pltpu.make_async_copy(src_hbm.at[base + j], sidx.at[b],
                                  semi[b]).wait()

        for t, tab in enumerate((tab0, tab1, tab2)):
            pltpu.sync_copy(zeros_hbm, acc.at[pl.ds(s * RT, RT)])
            plsc.subcore_barrier()

            def start_gather(j, b, tab=tab):
                pltpu.async_copy(tab.at[sidx.at[b]], bufs[b], semg[b])

            def wait_gather(b, tab=tab):
                pltpu.make_async_copy(tab.at[sidx.at[b]], bufs[b],
                                      semg[b]).wait()

            start_sidx(0, 0)
            start_sidx(1, 1)
            wait_sidx(0, 0)
            start_gather(0, 0)

            def step(j, b, guard2, guard1):
                wait_gather(b)
                pltpu.sync_copy(bufs[b], acc.at[dst_v.at[j]], add=True)
                if guard2:
                    @pl.when(j + 2 < n)
                    def _():
                        start_sidx(j + 2, b)
                else:
                    start_sidx(j + 2, b)
                if guard1:
                    @pl.when(j + 1 < n)
                    def _():
                        wait_sidx(j + 1, 1 - b)
                        start_gather(j + 1, 1 - b)
                else:
                    wait_sidx(j + 1, 1 - b)
                    start_gather(j + 1, 1 - b)

            def body(g, carry):
                step(2 * g, 0, guard2=True, guard1=False)
                step(2 * g + 1, 1, guard2=True, guard1=True)
                return carry

            lax.fori_loop(0, n // 2, body, 0)

            plsc.subcore_barrier()
            pltpu.sync_copy(acc.at[pl.ds(s * RT, RT)],
                            out_hbm.at[t, c, pl.ds(s * RT, RT)])

    @pl.when(c == 0)
    def _():
        run(NSLOW * s, NSLOW)

    @pl.when(c == 1)
    def _():
        run(16 * NSLOW + NFAST * s, NFAST)


def _dinv_block(d_ref):
    deg = d_ref[0][:, 0:1] + d_ref[1][:, 0:1] + 1.0
    return lax.rsqrt(deg), jnp.sqrt(deg)


def _tc1_body(x_ref, w_ref, d_ref, o0_ref, o1_ref, o2_ref):
    dinv, _ = _dinv_block(d_ref)
    hp = jnp.dot(x_ref[...], w_ref[...],
                 preferred_element_type=jnp.float32) * dinv
    o0_ref[...] = hp[:, :D]
    o1_ref[...] = hp[:, D:2 * D]
    o2_ref[...] = hp[:, 2 * D:]


def _combine(s_ref, h_refs, d_ref, b_refs):
    dinv, sdeg = _dinv_block(d_ref)
    outs = []
    for t, (h_ref, b_ref) in enumerate(zip(h_refs, b_refs)):
        hp = h_ref[...]
        p = dinv * (s_ref[t, 0] + s_ref[t, 1] + hp)
        if t == 0:
            outs.append(p + b_ref[...])
        elif t == 1:
            outs.append(sdeg * hp - p + b_ref[...])
        else:
            outs.append(sdeg * hp + p + b_ref[...])
    return outs, dinv


def _tc2_body(s_ref, h0_ref, h1_ref, h2_ref, d_ref, b0_ref, b1_ref, b2_ref,
              w0_ref, w1_ref, w2_ref, o0_ref, o1_ref, o2_ref):
    cs, dinv = _combine(s_ref, (h0_ref, h1_ref, h2_ref), d_ref,
                        (b0_ref, b1_ref, b2_ref))
    for cmb, w_ref, o_ref in zip(cs, (w0_ref, w1_ref, w2_ref),
                                 (o0_ref, o1_ref, o2_ref)):
        o_ref[...] = jnp.dot(jnp.maximum(cmb, 0.0), w_ref[...],
                             preferred_element_type=jnp.float32) * dinv


def _tc3_body(s_ref, h0_ref, h1_ref, h2_ref, d_ref, b0_ref, b1_ref, b2_ref,
              w_ref, lb_ref, o_ref):
    cs, _ = _combine(s_ref, (h0_ref, h1_ref, h2_ref), d_ref,
                     (b0_ref, b1_ref, b2_ref))
    acc = lb_ref[...].astype(jnp.float32)
    for t, cmb in enumerate(cs):
        acc = acc + jnp.dot(cmb, w_ref[pl.ds(t * D, D), :],
                            preferred_element_type=jnp.float32)
    o_ref[...] = acc


_spec_rows = pl.BlockSpec((BR, D), lambda i: (i, 0))
_spec_full = lambda a, b: pl.BlockSpec((a, b), lambda i: (0, 0))
_spec_deg = pl.BlockSpec((2, BR, 16), lambda i: (0, i, 0))
_spec_s = pl.BlockSpec((3, 2, BR, D), lambda i: (0, 0, i, 0))
_out_rows = jax.ShapeDtypeStruct((NP, D), jnp.float32)

_tc1 = pl.pallas_call(
    _tc1_body,
    grid=(NP // BR,),
    in_specs=[_spec_rows, _spec_full(D, F), _spec_deg],
    out_specs=[_spec_rows] * 3,
    out_shape=[_out_rows] * 3,
)

_tc2 = pl.pallas_call(
    _tc2_body,
    grid=(NP // BR,),
    in_specs=[_spec_s, _spec_rows, _spec_rows, _spec_rows, _spec_deg]
             + [_spec_full(1, D)] * 3 + [_spec_full(D, D)] * 3,
    out_specs=[_spec_rows] * 3,
    out_shape=[_out_rows] * 3,
)

_tc3 = pl.pallas_call(
    _tc3_body,
    grid=(NP // BR,),
    in_specs=[_spec_s, _spec_rows, _spec_rows, _spec_rows, _spec_deg]
             + [_spec_full(1, D)] * 3 + [_spec_full(F, D), _spec_full(1, D)],
    out_specs=_spec_rows,
    out_shape=_out_rows,
)


def kernel(x, edge_index, W1_0, b1_0, W2_0, b2_0, W1_1, b1_1, W2_1, b2_1,
           W1_2, b1_2, W2_2, b2_2, lin_W, lin_b):
    f32 = jnp.float32
    W1f = jnp.concatenate([W1_0, W1_1, W1_2], axis=1)
    b1s = [b.reshape(1, D) for b in (b1_0, b1_1, b1_2)]
    b2s = [b.reshape(1, D) for b in (b2_0, b2_1, b2_2)]

    xp = jnp.pad(x, ((0, NP - N), (0, 0)))
    src = jnp.concatenate([edge_index[0], jnp.zeros((EP - E,), jnp.int32)])
    dst = jnp.concatenate([edge_index[1], jnp.full((EP - E,), N, jnp.int32)])
    src_m = src.reshape(NCHUNKS, CH)
    dst_m = dst.reshape(NCHUNKS, CH)
    dst_d = dst[:32 * NCH * CH].reshape(32, NCH, CH)

    ones16 = jnp.ones((CH, 16), f32)
    zdeg = jnp.zeros((RT, 16), f32)
    zacc = jnp.zeros((RT, D), f32)

    degacc = _deg_kernel(dst_d, ones16, zdeg)
    h1a, h1b, h1c = _tc1(xp, W1f, degacc)
    s1 = _scatter_kernel(h1a, h1b, h1c, src_m, dst_m, zacc)
    h2a, h2b, h2c = _tc2(s1, h1a, h1b, h1c, degacc, *b1s, W2_0, W2_1, W2_2)
    s2 = _scatter_kernel(h2a, h2b, h2c, src_m, dst_m, zacc)
    out = _tc3(s2, h2a, h2b, h2c, degacc, *b2s, lin_W, lin_b.reshape(1, D))
    return out[:N]


# speedup vs baseline: 1.0813x; 1.0813x over previous
import functools

import jax
import jax.numpy as jnp
from jax import lax
from jax.experimental import pallas as pl
from jax.experimental.pallas import tpu as pltpu
from jax.experimental.pallas import tpu_sc as plsc

N = 10000
E = 320000
D = 128
F = 384
NP = 10240
RT = NP // 16
CH = 128
NCH = 79
NFAST = 80
NSLOW = 80
NCHUNKS = 16 * (NFAST + NSLOW)
EP = NCHUNKS * CH
BR = 256

_MESH = plsc.VectorSubcoreMesh(core_axis_name="c", subcore_axis_name="s")


@functools.partial(
    pl.kernel,
    mesh=_MESH,
    out_type=jax.ShapeDtypeStruct((2, NP, 16), jnp.float32),
    scratch_types=[
        pltpu.VMEM((NCH, CH), jnp.int32),
        pltpu.VMEM((CH, 16), jnp.float32),
        pltpu.VMEM_SHARED((NP, 16), jnp.float32),
    ],
    compiler_params=pltpu.CompilerParams(use_tc_tiling_on_sc=False),
)
def _deg_kernel(dst_hbm, ones_hbm, zeros_hbm, out_hbm, dst_v, ones_v, acc):
    c = lax.axis_index("c")
    s = lax.axis_index("s")
    wid = c * 16 + s
    pltpu.sync_copy(zeros_hbm, acc.at[pl.ds(s * RT, RT)])
    pltpu.sync_copy(dst_hbm.at[wid], dst_v)
    pltpu.sync_copy(ones_hbm, ones_v)
    plsc.subcore_barrier()

    def body(j, carry):
        pltpu.sync_copy(ones_v, acc.at[dst_v.at[j]], add=True)
        return carry

    lax.fori_loop(0, NCH, body, 0)
    plsc.subcore_barrier()
    pltpu.sync_copy(acc.at[pl.ds(s * RT, RT)], out_hbm.at[c, pl.ds(s * RT, RT)])


@functools.partial(
    pl.kernel,
    mesh=_MESH,
    out_type=jax.ShapeDtypeStruct((3, 2, NP, D), jnp.float32),
    scratch_types=[
        pltpu.VMEM((2, CH), jnp.int32),
        pltpu.VMEM((NFAST, CH), jnp.int32),
        pltpu.VMEM((CH, D), jnp.float32),
        pltpu.VMEM((CH, D), jnp.float32),
        pltpu.SemaphoreType.DMA,
        pltpu.SemaphoreType.DMA,
        pltpu.SemaphoreType.DMA,
        pltpu.SemaphoreType.DMA,
        pltpu.VMEM_SHARED((NP, D), jnp.float32),
    ],
)
def _scatter_kernel(tab0, tab1, tab2, src_hbm, dst_hbm, zeros_hbm, out_hbm,
                    sidx, dst_v, bufa, bufb, semg0, semg1, semi0, semi1, acc):
    c = lax.axis_index("c")
    s = lax.axis_index("s")
    bufs = (bufa, bufb)
    semg = (semg0, semg1)
    semi = (semi0, semi1)

    def run(base, n):
        pltpu.sync_copy(dst_hbm.at[pl.ds(base, n)], dst_v.at[pl.ds(0, n)])

        def start_sidx(j, b):
            pltpu.async_copy(src_hbm.at[base + j], sidx.at[b], semi[b])

        def wait_sidx(j, b):
            pltpu.make_async_copy(src_hbm.at[base + j], sidx.at[b],
                                  semi[b]).wait()

        for t, tab in enumerate((tab0, tab1, tab2)):
            pltpu.sync_copy(zeros_hbm, acc.at[pl.ds(s * RT, RT)])
            plsc.subcore_barrier()

            def start_gather(j, b, tab=tab):
                pltpu.async_copy(tab.at[sidx.at[b]], bufs[b], semg[b])

            def wait_gather(b, tab=tab):
                pltpu.make_async_copy(tab.at[sidx.at[b]], bufs[b],
                                      semg[b]).wait()

            start_sidx(0, 0)
            start_sidx(1, 1)
            wait_sidx(0, 0)
            start_gather(0, 0)

            def step(j, b, guard2, guard1):
                wait_gather(b)
                pltpu.sync_copy(bufs[b], acc.at[dst_v.at[j]], add=True)
                if guard2:
                    @pl.when(j + 2 < n)
                    def _():
                        start_sidx(j + 2, b)
                else:
                    start_sidx(j + 2, b)
                if guard1:
                    @pl.when(j + 1 < n)
                    def _():
                        wait_sidx(j + 1, 1 - b)
                        start_gather(j + 1, 1 - b)
                else:
                    wait_sidx(j + 1, 1 - b)
                    start_gather(j + 1, 1 - b)

            def body(g, carry):
                step(2 * g, 0, guard2=True, guard1=False)
                step(2 * g + 1, 1, guard2=True, guard1=True)
                return carry

            lax.fori_loop(0, n // 2, body, 0)

            plsc.subcore_barrier()
            pltpu.sync_copy(acc.at[pl.ds(s * RT, RT)],
                            out_hbm.at[t, c, pl.ds(s * RT, RT)])

    @pl.when(c == 0)
    def _():
        run(NSLOW * s, NSLOW)

    @pl.when(c == 1)
    def _():
        run(16 * NSLOW + NFAST * s, NFAST)


def _dinv_block(d_ref):
    deg = d_ref[0][:, 0:1] + d_ref[1][:, 0:1] + 1.0
    return lax.rsqrt(deg), jnp.sqrt(deg)


def _tc1_body(x_ref, w_ref, d_ref, o0_ref, o1_ref, o2_ref):
    dinv, _ = _dinv_block(d_ref)
    hp = jnp.dot(x_ref[...], w_ref[...],
                 preferred_element_type=jnp.float32) * dinv
    o0_ref[...] = hp[:, :D]
    o1_ref[...] = hp[:, D:2 * D]
    o2_ref[...] = hp[:, 2 * D:]


def _combine(s_ref, h_refs, d_ref, b_refs):
    dinv, sdeg = _dinv_block(d_ref)
    outs = []
    for t, (h_ref, b_ref) in enumerate(zip(h_refs, b_refs)):
        hp = h_ref[...]
        p = dinv * (s_ref[t, 0] + s_ref[t, 1] + hp)
        if t == 0:
            outs.append(p + b_ref[...])
        elif t == 1:
            outs.append(sdeg * hp - p + b_ref[...])
        else:
            outs.append(sdeg * hp + p + b_ref[...])
    return outs, dinv


def _tc2_body(s_ref, h0_ref, h1_ref, h2_ref, d_ref, b0_ref, b1_ref, b2_ref,
              w0_ref, w1_ref, w2_ref, o0_ref, o1_ref, o2_ref):
    cs, dinv = _combine(s_ref, (h0_ref, h1_ref, h2_ref), d_ref,
                        (b0_ref, b1_ref, b2_ref))
    for cmb, w_ref, o_ref in zip(cs, (w0_ref, w1_ref, w2_ref),
                                 (o0_ref, o1_ref, o2_ref)):
        o_ref[...] = jnp.dot(jnp.maximum(cmb, 0.0), w_ref[...],
                             preferred_element_type=jnp.float32) * dinv


def _tc3_body(s_ref, h0_ref, h1_ref, h2_ref, d_ref, b0_ref, b1_ref, b2_ref,
              w_ref, lb_ref, o_ref):
    cs, _ = _combine(s_ref, (h0_ref, h1_ref, h2_ref), d_ref,
                     (b0_ref, b1_ref, b2_ref))
    acc = lb_ref[...].astype(jnp.float32)
    for t, cmb in enumerate(cs):
        acc = acc + jnp.dot(cmb, w_ref[pl.ds(t * D, D), :],
                            preferred_element_type=jnp.float32)
    o_ref[...] = acc


_spec_rows = pl.BlockSpec((BR, D), lambda i: (i, 0))
_spec_full = lambda a, b: pl.BlockSpec((a, b), lambda i: (0, 0))
_spec_deg = pl.BlockSpec((2, BR, 16), lambda i: (0, i, 0))
_spec_s = pl.BlockSpec((3, 2, BR, D), lambda i: (0, 0, i, 0))
_out_rows = jax.ShapeDtypeStruct((NP, D), jnp.float32)

_tc1 = pl.pallas_call(
    _tc1_body,
    grid=(NP // BR,),
    in_specs=[_spec_rows, _spec_full(D, F), _spec_deg],
    out_specs=[_spec_rows] * 3,
    out_shape=[_out_rows] * 3,
)

_tc2 = pl.pallas_call(
    _tc2_body,
    grid=(NP // BR,),
    in_specs=[_spec_s, _spec_rows, _spec_rows, _spec_rows, _spec_deg]
             + [_spec_full(1, D)] * 3 + [_spec_full(D, D)] * 3,
    out_specs=[_spec_rows] * 3,
    out_shape=[_out_rows] * 3,
)

_tc3 = pl.pallas_call(
    _tc3_body,
    grid=(NP // BR,),
    in_specs=[_spec_s, _spec_rows, _spec_rows, _spec_rows, _spec_deg]
             + [_spec_full(1, D)] * 3 + [_spec_full(F, D), _spec_full(1, D)],
    out_specs=_spec_rows,
    out_shape=_out_rows,
)


def kernel(x, edge_index, W1_0, b1_0, W2_0, b2_0, W1_1, b1_1, W2_1, b2_1,
           W1_2, b1_2, W2_2, b2_2, lin_W, lin_b):
    f32 = jnp.float32
    W1f = jnp.concatenate([W1_0, W1_1, W1_2], axis=1)
    b1s = [b.reshape(1, D) for b in (b1_0, b1_1, b1_2)]
    b2s = [b.reshape(1, D) for b in (b2_0, b2_1, b2_2)]

    xp = jnp.pad(x, ((0, NP - N), (0, 0)))
    pad_dst = N + (jnp.arange(EP - E, dtype=jnp.int32) % (NP - N))
    src = jnp.concatenate([edge_index[0], jnp.zeros((EP - E,), jnp.int32)])
    dst = jnp.concatenate([edge_index[1], pad_dst])
    src_m = src.reshape(NCHUNKS, CH)
    dst_m = dst.reshape(NCHUNKS, CH)
    dst_d = dst[:32 * NCH * CH].reshape(32, NCH, CH)

    ones16 = jnp.ones((CH, 16), f32)
    zdeg = jnp.zeros((RT, 16), f32)
    zacc = jnp.zeros((RT, D), f32)

    degacc = _deg_kernel(dst_d, ones16, zdeg)
    h1a, h1b, h1c = _tc1(xp, W1f, degacc)
    s1 = _scatter_kernel(h1a, h1b, h1c, src_m, dst_m, zacc)
    h2a, h2b, h2c = _tc2(s1, h1a, h1b, h1c, degacc, *b1s, W2_0, W2_1, W2_2)
    s2 = _scatter_kernel(h2a, h2b, h2c, src_m, dst_m, zacc)
    out = _tc3(s2, h2a, h2b, h2c, degacc, *b2s, lin_W, lin_b.reshape(1, D))
    return out[:N]


# speedup vs baseline: 1.0822x; 1.0008x over previous
import functools

import jax
import jax.numpy as jnp
from jax import lax
from jax.experimental import pallas as pl
from jax.experimental.pallas import tpu as pltpu
from jax.experimental.pallas import tpu_sc as plsc

N = 10000
E = 320000
D = 128
F = 384
NP = 10240
RT = NP // 16
CH = 128
NCH = 79
NFAST = 80
NSLOW = 80
NCHUNKS = 16 * (NFAST + NSLOW)
EP = NCHUNKS * CH
BR = 256

_MESH = plsc.VectorSubcoreMesh(core_axis_name="c", subcore_axis_name="s")


@functools.partial(
    pl.kernel,
    mesh=_MESH,
    out_type=jax.ShapeDtypeStruct((2, NP, 16), jnp.float32),
    scratch_types=[
        pltpu.VMEM((NCH, CH), jnp.int32),
        pltpu.VMEM((CH, 16), jnp.float32),
        pltpu.VMEM_SHARED((NP, 16), jnp.float32),
    ],
    compiler_params=pltpu.CompilerParams(use_tc_tiling_on_sc=False),
)
def _deg_kernel(dst_hbm, ones_hbm, zeros_hbm, out_hbm, dst_v, ones_v, acc):
    c = lax.axis_index("c")
    s = lax.axis_index("s")
    wid = c * 16 + s
    pltpu.sync_copy(zeros_hbm, acc.at[pl.ds(s * RT, RT)])
    pltpu.sync_copy(dst_hbm.at[wid], dst_v)
    pltpu.sync_copy(ones_hbm, ones_v)
    plsc.subcore_barrier()

    def body(j, carry):
        pltpu.sync_copy(ones_v, acc.at[dst_v.at[j]], add=True)
        return carry

    lax.fori_loop(0, NCH, body, 0)
    plsc.subcore_barrier()
    pltpu.sync_copy(acc.at[pl.ds(s * RT, RT)], out_hbm.at[c, pl.ds(s * RT, RT)])


@functools.partial(
    pl.kernel,
    mesh=_MESH,
    out_type=jax.ShapeDtypeStruct((3, 2, NP, D), jnp.float32),
    scratch_types=[
        pltpu.VMEM((2, CH), jnp.int32),
        pltpu.VMEM((NFAST, CH), jnp.int32),
        pltpu.VMEM((CH, D), jnp.float32),
        pltpu.VMEM((CH, D), jnp.float32),
        pltpu.SemaphoreType.DMA,
        pltpu.SemaphoreType.DMA,
        pltpu.SemaphoreType.DMA,
        pltpu.SemaphoreType.DMA,
        pltpu.VMEM_SHARED((NP, D), jnp.float32),
    ],
)
def _scatter_kernel(tab0, tab1, tab2, src_hbm, dst_hbm, zeros_hbm, out_hbm,
                    sidx, dst_v, bufa, bufb, semg0, semg1, semi0, semi1, acc):
    c = lax.axis_index("c")
    s = lax.axis_index("s")
    bufs = (bufa, bufb)
    semg = (semg0, semg1)
    semi = (semi0, semi1)

    def run(base, n):
        pltpu.sync_copy(dst_hbm.at[pl.ds(base, n)], dst_v.at[pl.ds(0, n)])

        def start_sidx(j, b):
            pltpu.async_copy(src_hbm.at[base + j], sidx.at[b], semi[b])

        def wait_sidx(j, b):
            pltpu.make_async_copy(src_hbm.at[base + j], sidx.at[b],
                                  semi[b]).wait()

        for t, tab in enumerate((tab0, tab1, tab2)):
            pltpu.sync_copy(zeros_hbm, acc.at[pl.ds(s * RT, RT)])
            plsc.subcore_barrier()

            def start_gather(j, b, tab=tab):
                pltpu.async_copy(tab.at[sidx.at[b]], bufs[b], semg[b])

            def wait_gather(b, tab=tab):
                pltpu.make_async_copy(tab.at[sidx.at[b]], bufs[b],
                                      semg[b]).wait()

            start_sidx(0, 0)
            start_sidx(1, 1)
            wait_sidx(0, 0)
            start_gather(0, 0)

            def step(j, b, guard2, guard1):
                wait_gather(b)
                pltpu.sync_copy(bufs[b], acc.at[dst_v.at[j]], add=True)
                if guard2:
                    @pl.when(j + 2 < n)
                    def _():
                        start_sidx(j + 2, b)
                else:
                    start_sidx(j + 2, b)
                if guard1:
                    @pl.when(j + 1 < n)
                    def _():
                        wait_sidx(j + 1, 1 - b)
                        start_gather(j + 1, 1 - b)
                else:
                    wait_sidx(j + 1, 1 - b)
                    start_gather(j + 1, 1 - b)

            def body(g, carry):
                step(2 * g, 0, guard2=True, guard1=False)
                step(2 * g + 1, 1, guard2=True, guard1=True)
                return carry

            lax.fori_loop(0, n // 2, body, 0)

            plsc.subcore_barrier()
            pltpu.sync_copy(acc.at[pl.ds(s * RT, RT)],
                            out_hbm.at[t, c, pl.ds(s * RT, RT)])

    run((c * 16 + s) * NFAST, NFAST)


def _dinv_block(d_ref):
    deg = d_ref[0][:, 0:1] + d_ref[1][:, 0:1] + 1.0
    return lax.rsqrt(deg), jnp.sqrt(deg)


def _tc1_body(x_ref, w_ref, d_ref, o0_ref, o1_ref, o2_ref):
    dinv, _ = _dinv_block(d_ref)
    hp = jnp.dot(x_ref[...], w_ref[...],
                 preferred_element_type=jnp.float32) * dinv
    o0_ref[...] = hp[:, :D]
    o1_ref[...] = hp[:, D:2 * D]
    o2_ref[...] = hp[:, 2 * D:]


def _combine(s_ref, h_refs, d_ref, b_refs):
    dinv, sdeg = _dinv_block(d_ref)
    outs = []
    for t, (h_ref, b_ref) in enumerate(zip(h_refs, b_refs)):
        hp = h_ref[...]
        p = dinv * (s_ref[t, 0] + s_ref[t, 1] + hp)
        if t == 0:
            outs.append(p + b_ref[...])
        elif t == 1:
            outs.append(sdeg * hp - p + b_ref[...])
        else:
            outs.append(sdeg * hp + p + b_ref[...])
    return outs, dinv


def _tc2_body(s_ref, h0_ref, h1_ref, h2_ref, d_ref, b0_ref, b1_ref, b2_ref,
              w0_ref, w1_ref, w2_ref, o0_ref, o1_ref, o2_ref):
    cs, dinv = _combine(s_ref, (h0_ref, h1_ref, h2_ref), d_ref,
                        (b0_ref, b1_ref, b2_ref))
    for cmb, w_ref, o_ref in zip(cs, (w0_ref, w1_ref, w2_ref),
                                 (o0_ref, o1_ref, o2_ref)):
        o_ref[...] = jnp.dot(jnp.maximum(cmb, 0.0), w_ref[...],
                             preferred_element_type=jnp.float32) * dinv


def _tc3_body(s_ref, h0_ref, h1_ref, h2_ref, d_ref, b0_ref, b1_ref, b2_ref,
              w_ref, lb_ref, o_ref):
    cs, _ = _combine(s_ref, (h0_ref, h1_ref, h2_ref), d_ref,
                     (b0_ref, b1_ref, b2_ref))
    acc = lb_ref[...].astype(jnp.float32)
    for t, cmb in enumerate(cs):
        acc = acc + jnp.dot(cmb, w_ref[pl.ds(t * D, D), :],
                            preferred_element_type=jnp.float32)
    o_ref[...] = acc


_spec_rows = pl.BlockSpec((BR, D), lambda i: (i, 0))
_spec_full = lambda a, b: pl.BlockSpec((a, b), lambda i: (0, 0))
_spec_deg = pl.BlockSpec((2, BR, 16), lambda i: (0, i, 0))
_spec_s = pl.BlockSpec((3, 2, BR, D), lambda i: (0, 0, i, 0))
_out_rows = jax.ShapeDtypeStruct((NP, D), jnp.float32)

_tc1 = pl.pallas_call(
    _tc1_body,
    grid=(NP // BR,),
    in_specs=[_spec_rows, _spec_full(D, F), _spec_deg],
    out_specs=[_spec_rows] * 3,
    out_shape=[_out_rows] * 3,
)

_tc2 = pl.pallas_call(
    _tc2_body,
    grid=(NP // BR,),
    in_specs=[_spec_s, _spec_rows, _spec_rows, _spec_rows, _spec_deg]
             + [_spec_full(1, D)] * 3 + [_spec_full(D, D)] * 3,
    out_specs=[_spec_rows] * 3,
    out_shape=[_out_rows] * 3,
)

_tc3 = pl.pallas_call(
    _tc3_body,
    grid=(NP // BR,),
    in_specs=[_spec_s, _spec_rows, _spec_rows, _spec_rows, _spec_deg]
             + [_spec_full(1, D)] * 3 + [_spec_full(F, D), _spec_full(1, D)],
    out_specs=_spec_rows,
    out_shape=_out_rows,
)


def kernel(x, edge_index, W1_0, b1_0, W2_0, b2_0, W1_1, b1_1, W2_1, b2_1,
           W1_2, b1_2, W2_2, b2_2, lin_W, lin_b):
    f32 = jnp.float32
    W1f = jnp.concatenate([W1_0, W1_1, W1_2], axis=1)
    b1s = [b.reshape(1, D) for b in (b1_0, b1_1, b1_2)]
    b2s = [b.reshape(1, D) for b in (b2_0, b2_1, b2_2)]

    xp = jnp.pad(x, ((0, NP - N), (0, 0)))
    pad_dst = N + (jnp.arange(EP - E, dtype=jnp.int32) % (NP - N))
    src = jnp.concatenate([edge_index[0], jnp.zeros((EP - E,), jnp.int32)])
    dst = jnp.concatenate([edge_index[1], pad_dst])
    src_m = src.reshape(NCHUNKS, CH)
    dst_m = dst.reshape(NCHUNKS, CH)
    dst_d = dst[:32 * NCH * CH].reshape(32, NCH, CH)

    ones16 = jnp.ones((CH, 16), f32)
    zdeg = jnp.zeros((RT, 16), f32)
    zacc = jnp.zeros((RT, D), f32)

    degacc = _deg_kernel(dst_d, ones16, zdeg)
    h1a, h1b, h1c = _tc1(xp, W1f, degacc)
    s1 = _scatter_kernel(h1a, h1b, h1c, src_m, dst_m, zacc)
    h2a, h2b, h2c = _tc2(s1, h1a, h1b, h1c, degacc, *b1s, W2_0, W2_1, W2_2)
    s2 = _scatter_kernel(h2a, h2b, h2c, src_m, dst_m, zacc)
    out = _tc3(s2, h2a, h2b, h2c, degacc, *b2s, lin_W, lin_b.reshape(1, D))
    return out[:N]


# speedup vs baseline: 2.8350x; 2.6197x over previous
import functools

import jax
import jax.numpy as jnp
from jax import lax
from jax.experimental import pallas as pl
from jax.experimental.pallas import tpu as pltpu
from jax.experimental.pallas import tpu_sc as plsc

N = 10000
E = 320000
D = 128
F = 384
NP = 10240
RT = NP // 16
CH = 128
NCH = 79
NFAST = 80
NSLOW = 80
NCHUNKS = 16 * (NFAST + NSLOW)
EP = NCHUNKS * CH
BR = 256

_MESH = plsc.VectorSubcoreMesh(core_axis_name="c", subcore_axis_name="s")


@functools.partial(
    pl.kernel,
    mesh=_MESH,
    out_type=jax.ShapeDtypeStruct((2, NP, 16), jnp.float32),
    scratch_types=[
        pltpu.VMEM((NCH, CH), jnp.int32),
        pltpu.VMEM((CH, 16), jnp.float32),
        pltpu.VMEM_SHARED((NP, 16), jnp.float32),
    ],
    compiler_params=pltpu.CompilerParams(use_tc_tiling_on_sc=False),
)
def _deg_kernel(dst_hbm, ones_hbm, zeros_hbm, out_hbm, dst_v, ones_v, acc):
    c = lax.axis_index("c")
    s = lax.axis_index("s")
    wid = c * 16 + s
    pltpu.sync_copy(zeros_hbm, acc.at[pl.ds(s * RT, RT)])
    pltpu.sync_copy(dst_hbm.at[wid], dst_v)
    pltpu.sync_copy(ones_hbm, ones_v)
    plsc.subcore_barrier()

    def body(j, carry):
        pltpu.sync_copy(ones_v, acc.at[dst_v.at[j]], add=True)
        return carry

    lax.fori_loop(0, NCH, body, 0)
    plsc.subcore_barrier()
    pltpu.sync_copy(acc.at[pl.ds(s * RT, RT)], out_hbm.at[c, pl.ds(s * RT, RT)])


@functools.partial(
    pl.kernel,
    mesh=_MESH,
    out_type=jax.ShapeDtypeStruct((3, 2, NP, D), jnp.float32),
    scratch_types=[
        pltpu.VMEM((2, CH), jnp.int32),
        pltpu.VMEM((NFAST, CH), jnp.int32),
        pltpu.VMEM((CH, D), jnp.float32),
        pltpu.VMEM((CH, D), jnp.float32),
        pltpu.SemaphoreType.DMA,
        pltpu.SemaphoreType.DMA,
        pltpu.SemaphoreType.DMA,
        pltpu.SemaphoreType.DMA,
        pltpu.VMEM_SHARED((NP, D), jnp.float32),
    ],
)
def _scatter_kernel(tab0, tab1, tab2, src_hbm, dst_hbm, zeros_hbm, out_hbm,
                    sidx, dst_v, bufa, bufb, semg0, semg1, semi0, semi1, acc):
    c = lax.axis_index("c")
    s = lax.axis_index("s")
    bufs = (bufa, bufb)
    semg = (semg0, semg1)
    semi = (semi0, semi1)

    def run(base, n):
        pltpu.sync_copy(dst_hbm.at[pl.ds(base, n)], dst_v.at[pl.ds(0, n)])

        def start_sidx(j, b):
            pltpu.async_copy(src_hbm.at[base + j], sidx.at[b], semi[b])

        def wait_sidx(j, b):
            pltpu.make_async_copy(src_hbm.at[base + j], sidx.at[b],
                                  semi[b]).wait()

        for t, tab in enumerate((tab0, tab1, tab2)):
            pltpu.sync_copy(zeros_hbm, acc.at[pl.ds(s * RT, RT)])
            plsc.subcore_barrier()

            def start_gather(j, b, tab=tab):
                pltpu.async_copy(tab.at[sidx.at[b]], bufs[b], semg[b])

            def wait_gather(b, tab=tab):
                pltpu.make_async_copy(tab.at[sidx.at[b]], bufs[b],
                                      semg[b]).wait()

            start_sidx(0, 0)
            start_sidx(1, 1)
            wait_sidx(0, 0)
            start_gather(0, 0)

            def step(j, b, guard2, guard1):
                wait_gather(b)
                pltpu.sync_copy(bufs[b], acc.at[dst_v.at[j]], add=True)
                if guard2:
                    @pl.when(j + 2 < n)
                    def _():
                        start_sidx(j + 2, b)
                else:
                    start_sidx(j + 2, b)
                if guard1:
                    @pl.when(j + 1 < n)
                    def _():
                        wait_sidx(j + 1, 1 - b)
                        start_gather(j + 1, 1 - b)
                else:
                    wait_sidx(j + 1, 1 - b)
                    start_gather(j + 1, 1 - b)

            def body(g, carry):
                step(2 * g, 0, guard2=True, guard1=False)
                step(2 * g + 1, 1, guard2=True, guard1=True)
                return carry

            lax.fori_loop(0, n // 2, body, 0)

            plsc.subcore_barrier()
            pltpu.sync_copy(acc.at[pl.ds(s * RT, RT)],
                            out_hbm.at[t, c, pl.ds(s * RT, RT)])

    run((c * 16 + s) * NFAST, NFAST)


def _dinv_block(d_ref):
    deg = d_ref[0][:, 0:1] + d_ref[1][:, 0:1] + 1.0
    return lax.rsqrt(deg), jnp.sqrt(deg)


def _tc1_body(x_ref, w_ref, d_ref, o0_ref, o1_ref, o2_ref):
    dinv, _ = _dinv_block(d_ref)
    hp = jnp.dot(x_ref[...], w_ref[...],
                 preferred_element_type=jnp.float32) * dinv
    o0_ref[...] = hp[:, :D]
    o1_ref[...] = hp[:, D:2 * D]
    o2_ref[...] = hp[:, 2 * D:]


def _combine(s_ref, h_refs, d_ref, b_refs):
    dinv, sdeg = _dinv_block(d_ref)
    outs = []
    for t, (h_ref, b_ref) in enumerate(zip(h_refs, b_refs)):
        hp = h_ref[...]
        p = dinv * (s_ref[t, 0] + s_ref[t, 1] + hp)
        if t == 0:
            outs.append(p + b_ref[...])
        elif t == 1:
            outs.append(sdeg * hp - p + b_ref[...])
        else:
            outs.append(sdeg * hp + p + b_ref[...])
    return outs, dinv


def _tc2_body(s_ref, h0_ref, h1_ref, h2_ref, d_ref, b0_ref, b1_ref, b2_ref,
              w0_ref, w1_ref, w2_ref, o0_ref, o1_ref, o2_ref):
    cs, dinv = _combine(s_ref, (h0_ref, h1_ref, h2_ref), d_ref,
                        (b0_ref, b1_ref, b2_ref))
    for cmb, w_ref, o_ref in zip(cs, (w0_ref, w1_ref, w2_ref),
                                 (o0_ref, o1_ref, o2_ref)):
        o_ref[...] = jnp.dot(jnp.maximum(cmb, 0.0), w_ref[...],
                             preferred_element_type=jnp.float32) * dinv


def _tc3_body(s_ref, h0_ref, h1_ref, h2_ref, d_ref, b0_ref, b1_ref, b2_ref,
              w_ref, lb_ref, o_ref):
    cs, _ = _combine(s_ref, (h0_ref, h1_ref, h2_ref), d_ref,
                     (b0_ref, b1_ref, b2_ref))
    acc = lb_ref[...].astype(jnp.float32)
    for t, cmb in enumerate(cs):
        acc = acc + jnp.dot(cmb, w_ref[pl.ds(t * D, D), :],
                            preferred_element_type=jnp.float32)
    o_ref[...] = acc


_spec_rows = pl.BlockSpec((BR, D), lambda i: (i, 0))
_spec_full = lambda a, b: pl.BlockSpec((a, b), lambda i: (0, 0))
_spec_deg = pl.BlockSpec((2, BR, 16), lambda i: (0, i, 0))
_spec_s = pl.BlockSpec((3, 2, BR, D), lambda i: (0, 0, i, 0))
_out_rows = jax.ShapeDtypeStruct((NP, D), jnp.float32)

_tc1 = pl.pallas_call(
    _tc1_body,
    grid=(NP // BR,),
    in_specs=[_spec_rows, _spec_full(D, F), _spec_deg],
    out_specs=[_spec_rows] * 3,
    out_shape=[_out_rows] * 3,
)

_tc2 = pl.pallas_call(
    _tc2_body,
    grid=(NP // BR,),
    in_specs=[_spec_s, _spec_rows, _spec_rows, _spec_rows, _spec_deg]
             + [_spec_full(1, D)] * 3 + [_spec_full(D, D)] * 3,
    out_specs=[_spec_rows] * 3,
    out_shape=[_out_rows] * 3,
)

_tc3 = pl.pallas_call(
    _tc3_body,
    grid=(NP // BR,),
    in_specs=[_spec_s, _spec_rows, _spec_rows, _spec_rows, _spec_deg]
             + [_spec_full(1, D)] * 3 + [_spec_full(F, D), _spec_full(1, D)],
    out_specs=_spec_rows,
    out_shape=_out_rows,
)


def kernel(x, edge_index, W1_0, b1_0, W2_0, b2_0, W1_1, b1_1, W2_1, b2_1,
           W1_2, b1_2, W2_2, b2_2, lin_W, lin_b):
    f32 = jnp.float32
    W1f = jnp.concatenate([W1_0, W1_1, W1_2], axis=1)
    b1s = [b.reshape(1, D) for b in (b1_0, b1_1, b1_2)]
    b2s = [b.reshape(1, D) for b in (b2_0, b2_1, b2_2)]

    xp = jnp.pad(x, ((0, NP - N), (0, 0)))
    pad_dst = N + (jnp.arange(EP - E, dtype=jnp.int32) % (NP - N))
    pad_src = jnp.arange(EP - E, dtype=jnp.int32) % N
    src = jnp.concatenate([edge_index[0], pad_src])
    dst = jnp.concatenate([edge_index[1], pad_dst])
    src_m = src.reshape(NCHUNKS, CH)
    dst_m = dst.reshape(NCHUNKS, CH)
    dst_d = dst[:32 * NCH * CH].reshape(32, NCH, CH)

    ones16 = jnp.ones((CH, 16), f32)
    zdeg = jnp.zeros((RT, 16), f32)
    zacc = jnp.zeros((RT, D), f32)

    degacc = _deg_kernel(dst_d, ones16, zdeg)
    h1a, h1b, h1c = _tc1(xp, W1f, degacc)
    s1 = _scatter_kernel(h1a, h1b, h1c, src_m, dst_m, zacc)
    h2a, h2b, h2c = _tc2(s1, h1a, h1b, h1c, degacc, *b1s, W2_0, W2_1, W2_2)
    s2 = _scatter_kernel(h2a, h2b, h2c, src_m, dst_m, zacc)
    out = _tc3(s2, h2a, h2b, h2c, degacc, *b2s, lin_W, lin_b.reshape(1, D))
    return out[:N]


# speedup vs baseline: 3.6782x; 1.2974x over previous
import functools

import jax
import jax.numpy as jnp
from jax import lax
from jax.experimental import pallas as pl
from jax.experimental.pallas import tpu as pltpu
from jax.experimental.pallas import tpu_sc as plsc

N = 10000
E = 320000
D = 128
F = 384
NP = 10240
RT = NP // 16
CH = 128
NCH = 79
NFAST = 80
NSLOW = 80
NCHUNKS = 16 * (NFAST + NSLOW)
EP = NCHUNKS * CH
BR = 256

_MESH = plsc.VectorSubcoreMesh(core_axis_name="c", subcore_axis_name="s")


@functools.partial(
    pl.kernel,
    mesh=_MESH,
    out_type=jax.ShapeDtypeStruct((2, NP, 16), jnp.float32),
    scratch_types=[
        pltpu.VMEM((NCH, CH), jnp.int32),
        pltpu.VMEM((CH, 16), jnp.float32),
        pltpu.VMEM_SHARED((NP, 16), jnp.float32),
    ],
    compiler_params=pltpu.CompilerParams(use_tc_tiling_on_sc=False),
)
def _deg_kernel(dst_hbm, ones_hbm, zeros_hbm, out_hbm, dst_v, ones_v, acc):
    c = lax.axis_index("c")
    s = lax.axis_index("s")
    wid = c * 16 + s
    pltpu.sync_copy(zeros_hbm, acc.at[pl.ds(s * RT, RT)])
    pltpu.sync_copy(dst_hbm.at[wid], dst_v)
    pltpu.sync_copy(ones_hbm, ones_v)
    plsc.subcore_barrier()

    def body(j, carry):
        pltpu.sync_copy(ones_v, acc.at[dst_v.at[j]], add=True)
        return carry

    lax.fori_loop(0, NCH, body, 0)
    plsc.subcore_barrier()
    pltpu.sync_copy(acc.at[pl.ds(s * RT, RT)], out_hbm.at[c, pl.ds(s * RT, RT)])


@functools.partial(
    pl.kernel,
    mesh=_MESH,
    out_type=jax.ShapeDtypeStruct((3, 2, NP, D), jnp.float32),
    scratch_types=[
        pltpu.VMEM((2, CH), jnp.int32),
        pltpu.VMEM((NFAST, CH), jnp.int32),
        pltpu.VMEM((CH, D), jnp.float32),
        pltpu.VMEM((CH, D), jnp.float32),
        pltpu.SemaphoreType.DMA,
        pltpu.SemaphoreType.DMA,
        pltpu.SemaphoreType.DMA,
        pltpu.SemaphoreType.DMA,
        pltpu.SemaphoreType.DMA,
        pltpu.SemaphoreType.DMA,
        pltpu.VMEM_SHARED((NP, D), jnp.float32),
    ],
)
def _scatter_kernel(tab0, tab1, tab2, src_hbm, dst_hbm, zeros_hbm, out_hbm,
                    sidx, dst_v, bufa, bufb, semg0, semg1, semi0, semi1,
                    semsc0, semsc1, acc):
    c = lax.axis_index("c")
    s = lax.axis_index("s")
    bufs = (bufa, bufb)
    semg = (semg0, semg1)
    semi = (semi0, semi1)
    semsc = (semsc0, semsc1)

    def run(base, n):
        pltpu.sync_copy(dst_hbm.at[pl.ds(base, n)], dst_v.at[pl.ds(0, n)])

        def start_sidx(j, b):
            pltpu.async_copy(src_hbm.at[base + j], sidx.at[b], semi[b])

        def wait_sidx(j, b):
            pltpu.make_async_copy(src_hbm.at[base + j], sidx.at[b],
                                  semi[b]).wait()

        for t, tab in enumerate((tab0, tab1, tab2)):
            pltpu.sync_copy(zeros_hbm, acc.at[pl.ds(s * RT, RT)])
            plsc.subcore_barrier()

            def start_gather(j, b, tab=tab):
                pltpu.async_copy(tab.at[sidx.at[b]], bufs[b], semg[b])

            def wait_gather(b, tab=tab):
                pltpu.make_async_copy(tab.at[sidx.at[b]], bufs[b],
                                      semg[b]).wait()

            def start_scatter(j, b):
                pltpu.async_copy(bufs[b], acc.at[dst_v.at[j]], semsc[b],
                                 add=True)

            def wait_scatter(j, b):
                pltpu.make_async_copy(bufs[b], acc.at[dst_v.at[j]],
                                      semsc[b]).wait()

            start_sidx(0, 0)
            start_sidx(1, 1)
            wait_sidx(0, 0)
            start_gather(0, 0)

            def step(j, b, guard2, guard1, first=False):
                wait_gather(b)
                start_scatter(j, b)
                if guard2:
                    @pl.when(j + 2 < n)
                    def _():
                        start_sidx(j + 2, b)
                else:
                    start_sidx(j + 2, b)

                def advance():
                    wait_sidx(j + 1, 1 - b)
                    if first:
                        @pl.when(j > 0)
                        def _():
                            wait_scatter(j - 1, 1 - b)
                    else:
                        wait_scatter(j - 1, 1 - b)
                    start_gather(j + 1, 1 - b)

                if guard1:
                    @pl.when(j + 1 < n)
                    def _():
                        advance()
                else:
                    advance()

            def body(g, carry):
                step(2 * g, 0, guard2=True, guard1=False, first=True)
                step(2 * g + 1, 1, guard2=True, guard1=True)
                return carry

            lax.fori_loop(0, n // 2, body, 0)
            wait_scatter(n - 2, 0)
            wait_scatter(n - 1, 1)

            plsc.subcore_barrier()
            pltpu.sync_copy(acc.at[pl.ds(s * RT, RT)],
                            out_hbm.at[t, c, pl.ds(s * RT, RT)])

    run((c * 16 + s) * NFAST, NFAST)


def _dinv_block(d_ref):
    deg = d_ref[0][:, 0:1] + d_ref[1][:, 0:1] + 1.0
    return lax.rsqrt(deg), jnp.sqrt(deg)


def _tc1_body(x_ref, w_ref, d_ref, o0_ref, o1_ref, o2_ref):
    dinv, _ = _dinv_block(d_ref)
    hp = jnp.dot(x_ref[...], w_ref[...],
                 preferred_element_type=jnp.float32) * dinv
    o0_ref[...] = hp[:, :D]
    o1_ref[...] = hp[:, D:2 * D]
    o2_ref[...] = hp[:, 2 * D:]


def _combine(s_ref, h_refs, d_ref, b_refs):
    dinv, sdeg = _dinv_block(d_ref)
    outs = []
    for t, (h_ref, b_ref) in enumerate(zip(h_refs, b_refs)):
        hp = h_ref[...]
        p = dinv * (s_ref[t, 0] + s_ref[t, 1] + hp)
        if t == 0:
            outs.append(p + b_ref[...])
        elif t == 1:
            outs.append(sdeg * hp - p + b_ref[...])
        else:
            outs.append(sdeg * hp + p + b_ref[...])
    return outs, dinv


def _tc2_body(s_ref, h0_ref, h1_ref, h2_ref, d_ref, b0_ref, b1_ref, b2_ref,
              w0_ref, w1_ref, w2_ref, o0_ref, o1_ref, o2_ref):
    cs, dinv = _combine(s_ref, (h0_ref, h1_ref, h2_ref), d_ref,
                        (b0_ref, b1_ref, b2_ref))
    for cmb, w_ref, o_ref in zip(cs, (w0_ref, w1_ref, w2_ref),
                                 (o0_ref, o1_ref, o2_ref)):
        o_ref[...] = jnp.dot(jnp.maximum(cmb, 0.0), w_ref[...],
                             preferred_element_type=jnp.float32) * dinv


def _tc3_body(s_ref, h0_ref, h1_ref, h2_ref, d_ref, b0_ref, b1_ref, b2_ref,
              w_ref, lb_ref, o_ref):
    cs, _ = _combine(s_ref, (h0_ref, h1_ref, h2_ref), d_ref,
                     (b0_ref, b1_ref, b2_ref))
    acc = lb_ref[...].astype(jnp.float32)
    for t, cmb in enumerate(cs):
        acc = acc + jnp.dot(cmb, w_ref[pl.ds(t * D, D), :],
                            preferred_element_type=jnp.float32)
    o_ref[...] = acc


_spec_rows = pl.BlockSpec((BR, D), lambda i: (i, 0))
_spec_full = lambda a, b: pl.BlockSpec((a, b), lambda i: (0, 0))
_spec_deg = pl.BlockSpec((2, BR, 16), lambda i: (0, i, 0))
_spec_s = pl.BlockSpec((3, 2, BR, D), lambda i: (0, 0, i, 0))
_out_rows = jax.ShapeDtypeStruct((NP, D), jnp.float32)

_tc1 = pl.pallas_call(
    _tc1_body,
    grid=(NP // BR,),
    in_specs=[_spec_rows, _spec_full(D, F), _spec_deg],
    out_specs=[_spec_rows] * 3,
    out_shape=[_out_rows] * 3,
)

_tc2 = pl.pallas_call(
    _tc2_body,
    grid=(NP // BR,),
    in_specs=[_spec_s, _spec_rows, _spec_rows, _spec_rows, _spec_deg]
             + [_spec_full(1, D)] * 3 + [_spec_full(D, D)] * 3,
    out_specs=[_spec_rows] * 3,
    out_shape=[_out_rows] * 3,
)

_tc3 = pl.pallas_call(
    _tc3_body,
    grid=(NP // BR,),
    in_specs=[_spec_s, _spec_rows, _spec_rows, _spec_rows, _spec_deg]
             + [_spec_full(1, D)] * 3 + [_spec_full(F, D), _spec_full(1, D)],
    out_specs=_spec_rows,
    out_shape=_out_rows,
)


def kernel(x, edge_index, W1_0, b1_0, W2_0, b2_0, W1_1, b1_1, W2_1, b2_1,
           W1_2, b1_2, W2_2, b2_2, lin_W, lin_b):
    f32 = jnp.float32
    W1f = jnp.concatenate([W1_0, W1_1, W1_2], axis=1)
    b1s = [b.reshape(1, D) for b in (b1_0, b1_1, b1_2)]
    b2s = [b.reshape(1, D) for b in (b2_0, b2_1, b2_2)]

    xp = jnp.pad(x, ((0, NP - N), (0, 0)))
    pad_dst = N + (jnp.arange(EP - E, dtype=jnp.int32) % (NP - N))
    pad_src = jnp.arange(EP - E, dtype=jnp.int32) % N
    src = jnp.concatenate([edge_index[0], pad_src])
    dst = jnp.concatenate([edge_index[1], pad_dst])
    src_m = src.reshape(NCHUNKS, CH)
    dst_m = dst.reshape(NCHUNKS, CH)
    dst_d = dst[:32 * NCH * CH].reshape(32, NCH, CH)

    ones16 = jnp.ones((CH, 16), f32)
    zdeg = jnp.zeros((RT, 16), f32)
    zacc = jnp.zeros((RT, D), f32)

    degacc = _deg_kernel(dst_d, ones16, zdeg)
    h1a, h1b, h1c = _tc1(xp, W1f, degacc)
    s1 = _scatter_kernel(h1a, h1b, h1c, src_m, dst_m, zacc)
    h2a, h2b, h2c = _tc2(s1, h1a, h1b, h1c, degacc, *b1s, W2_0, W2_1, W2_2)
    s2 = _scatter_kernel(h2a, h2b, h2c, src_m, dst_m, zacc)
    out = _tc3(s2, h2a, h2b, h2c, degacc, *b2s, lin_W, lin_b.reshape(1, D))
    return out[:N]


# speedup vs baseline: 4.2885x; 1.1659x over previous
import functools

import jax
import jax.numpy as jnp
from jax import lax
from jax.experimental import pallas as pl
from jax.experimental.pallas import tpu as pltpu
from jax.experimental.pallas import tpu_sc as plsc

N = 10000
E = 320000
D = 128
F = 384
NP = 10240
RT = NP // 16
CH = 128
NCH = 79
NFAST = 80
NSLOW = 80
NCHUNKS = 16 * (NFAST + NSLOW)
EP = NCHUNKS * CH
BR = 256

_MESH = plsc.VectorSubcoreMesh(core_axis_name="c", subcore_axis_name="s")


@functools.partial(
    pl.kernel,
    mesh=_MESH,
    out_type=jax.ShapeDtypeStruct((2, NP, 16), jnp.float32),
    scratch_types=[
        pltpu.VMEM((NCH, CH), jnp.int32),
        pltpu.VMEM((CH, 16), jnp.float32),
        pltpu.VMEM_SHARED((NP, 16), jnp.float32),
    ],
    compiler_params=pltpu.CompilerParams(use_tc_tiling_on_sc=False),
)
def _deg_kernel(dst_hbm, ones_hbm, zeros_hbm, out_hbm, dst_v, ones_v, acc):
    c = lax.axis_index("c")
    s = lax.axis_index("s")
    wid = c * 16 + s
    pltpu.sync_copy(zeros_hbm, acc.at[pl.ds(s * RT, RT)])
    pltpu.sync_copy(dst_hbm.at[wid], dst_v)
    pltpu.sync_copy(ones_hbm, ones_v)
    plsc.subcore_barrier()

    def body(j, carry):
        pltpu.sync_copy(ones_v, acc.at[dst_v.at[j]], add=True)
        return carry

    lax.fori_loop(0, NCH, body, 0)
    plsc.subcore_barrier()
    pltpu.sync_copy(acc.at[pl.ds(s * RT, RT)], out_hbm.at[c, pl.ds(s * RT, RT)])


@functools.partial(
    pl.kernel,
    mesh=_MESH,
    out_type=jax.ShapeDtypeStruct((3, 2, NP, D), jnp.float32),
    scratch_types=[
        pltpu.VMEM((2, CH), jnp.int32),
        pltpu.VMEM((NFAST, CH), jnp.int32),
        pltpu.VMEM((CH, D), jnp.float32),
        pltpu.VMEM((CH, D), jnp.float32),
        pltpu.SemaphoreType.DMA,
        pltpu.SemaphoreType.DMA,
        pltpu.SemaphoreType.DMA,
        pltpu.SemaphoreType.DMA,
        pltpu.SemaphoreType.DMA,
        pltpu.SemaphoreType.DMA,
        pltpu.VMEM_SHARED((NP, D), jnp.float32),
    ],
)
def _scatter_kernel(tab0, tab1, tab2, src_hbm, dst_hbm, zeros_hbm, out_hbm,
                    sidx, dst_v, bufa, bufb, semg0, semg1, semi0, semi1,
                    semsc0, semsc1, acc):
    c = lax.axis_index("c")
    s = lax.axis_index("s")
    bufs = (bufa, bufb)
    semg = (semg0, semg1)
    semi = (semi0, semi1)
    semsc = (semsc0, semsc1)

    def run(base, n):
        pltpu.sync_copy(dst_hbm.at[pl.ds(base, n)], dst_v.at[pl.ds(0, n)])

        def start_sidx(j, b):
            pltpu.async_copy(src_hbm.at[base + j], sidx.at[b], semi[b])

        def wait_sidx(j, b):
            pltpu.make_async_copy(src_hbm.at[base + j], sidx.at[b],
                                  semi[b]).wait()

        for t, tab in enumerate((tab0, tab1, tab2)):
            pltpu.sync_copy(zeros_hbm, acc.at[pl.ds(s * RT, RT)])
            plsc.subcore_barrier()

            def start_gather(j, b, tab=tab):
                pltpu.async_copy(tab.at[sidx.at[b]], bufs[b], semg[b])

            def wait_gather(b, tab=tab):
                pltpu.make_async_copy(tab.at[sidx.at[b]], bufs[b],
                                      semg[b]).wait()

            def start_scatter(j, b):
                pltpu.async_copy(bufs[b], acc.at[dst_v.at[j]], semsc[b],
                                 add=True)

            def wait_scatter(j, b):
                pltpu.make_async_copy(bufs[b], acc.at[dst_v.at[j]],
                                      semsc[b]).wait()

            start_sidx(0, 0)
            start_sidx(1, 1)
            wait_sidx(0, 0)
            start_gather(0, 0)

            def step(j, b, guard2, guard1, first=False):
                def advance():
                    wait_sidx(j + 1, 1 - b)
                    if first:
                        @pl.when(j > 0)
                        def _():
                            wait_scatter(j - 1, 1 - b)
                    else:
                        wait_scatter(j - 1, 1 - b)
                    start_gather(j + 1, 1 - b)

                if guard1:
                    @pl.when(j + 1 < n)
                    def _():
                        advance()
                else:
                    advance()
                wait_gather(b)
                start_scatter(j, b)
                if guard2:
                    @pl.when(j + 2 < n)
                    def _():
                        start_sidx(j + 2, b)
                else:
                    start_sidx(j + 2, b)

            def body(g, carry):
                step(2 * g, 0, guard2=True, guard1=False, first=True)
                step(2 * g + 1, 1, guard2=True, guard1=True)
                return carry

            lax.fori_loop(0, n // 2, body, 0)
            wait_scatter(n - 2, 0)
            wait_scatter(n - 1, 1)

            plsc.subcore_barrier()
            pltpu.sync_copy(acc.at[pl.ds(s * RT, RT)],
                            out_hbm.at[t, c, pl.ds(s * RT, RT)])

    run((c * 16 + s) * NFAST, NFAST)


def _dinv_block(d_ref):
    deg = d_ref[0][:, 0:1] + d_ref[1][:, 0:1] + 1.0
    return lax.rsqrt(deg), jnp.sqrt(deg)


def _tc1_body(x_ref, w_ref, d_ref, o0_ref, o1_ref, o2_ref):
    dinv, _ = _dinv_block(d_ref)
    hp = jnp.dot(x_ref[...], w_ref[...],
                 preferred_element_type=jnp.float32) * dinv
    o0_ref[...] = hp[:, :D]
    o1_ref[...] = hp[:, D:2 * D]
    o2_ref[...] = hp[:, 2 * D:]


def _combine(s_ref, h_refs, d_ref, b_refs):
    dinv, sdeg = _dinv_block(d_ref)
    outs = []
    for t, (h_ref, b_ref) in enumerate(zip(h_refs, b_refs)):
        hp = h_ref[...]
        p = dinv * (s_ref[t, 0] + s_ref[t, 1] + hp)
        if t == 0:
            outs.append(p + b_ref[...])
        elif t == 1:
            outs.append(sdeg * hp - p + b_ref[...])
        else:
            outs.append(sdeg * hp + p + b_ref[...])
    return outs, dinv


def _tc2_body(s_ref, h0_ref, h1_ref, h2_ref, d_ref, b0_ref, b1_ref, b2_ref,
              w0_ref, w1_ref, w2_ref, o0_ref, o1_ref, o2_ref):
    cs, dinv = _combine(s_ref, (h0_ref, h1_ref, h2_ref), d_ref,
                        (b0_ref, b1_ref, b2_ref))
    for cmb, w_ref, o_ref in zip(cs, (w0_ref, w1_ref, w2_ref),
                                 (o0_ref, o1_ref, o2_ref)):
        o_ref[...] = jnp.dot(jnp.maximum(cmb, 0.0), w_ref[...],
                             preferred_element_type=jnp.float32) * dinv


def _tc3_body(s_ref, h0_ref, h1_ref, h2_ref, d_ref, b0_ref, b1_ref, b2_ref,
              w_ref, lb_ref, o_ref):
    cs, _ = _combine(s_ref, (h0_ref, h1_ref, h2_ref), d_ref,
                     (b0_ref, b1_ref, b2_ref))
    acc = lb_ref[...].astype(jnp.float32)
    for t, cmb in enumerate(cs):
        acc = acc + jnp.dot(cmb, w_ref[pl.ds(t * D, D), :],
                            preferred_element_type=jnp.float32)
    o_ref[...] = acc


_spec_rows = pl.BlockSpec((BR, D), lambda i: (i, 0))
_spec_full = lambda a, b: pl.BlockSpec((a, b), lambda i: (0, 0))
_spec_deg = pl.BlockSpec((2, BR, 16), lambda i: (0, i, 0))
_spec_s = pl.BlockSpec((3, 2, BR, D), lambda i: (0, 0, i, 0))
_out_rows = jax.ShapeDtypeStruct((NP, D), jnp.float32)

_tc1 = pl.pallas_call(
    _tc1_body,
    grid=(NP // BR,),
    in_specs=[_spec_rows, _spec_full(D, F), _spec_deg],
    out_specs=[_spec_rows] * 3,
    out_shape=[_out_rows] * 3,
)

_tc2 = pl.pallas_call(
    _tc2_body,
    grid=(NP // BR,),
    in_specs=[_spec_s, _spec_rows, _spec_rows, _spec_rows, _spec_deg]
             + [_spec_full(1, D)] * 3 + [_spec_full(D, D)] * 3,
    out_specs=[_spec_rows] * 3,
    out_shape=[_out_rows] * 3,
)

_tc3 = pl.pallas_call(
    _tc3_body,
    grid=(NP // BR,),
    in_specs=[_spec_s, _spec_rows, _spec_rows, _spec_rows, _spec_deg]
             + [_spec_full(1, D)] * 3 + [_spec_full(F, D), _spec_full(1, D)],
    out_specs=_spec_rows,
    out_shape=_out_rows,
)


def kernel(x, edge_index, W1_0, b1_0, W2_0, b2_0, W1_1, b1_1, W2_1, b2_1,
           W1_2, b1_2, W2_2, b2_2, lin_W, lin_b):
    f32 = jnp.float32
    W1f = jnp.concatenate([W1_0, W1_1, W1_2], axis=1)
    b1s = [b.reshape(1, D) for b in (b1_0, b1_1, b1_2)]
    b2s = [b.reshape(1, D) for b in (b2_0, b2_1, b2_2)]

    xp = jnp.pad(x, ((0, NP - N), (0, 0)))
    pad_dst = N + (jnp.arange(EP - E, dtype=jnp.int32) % (NP - N))
    pad_src = jnp.arange(EP - E, dtype=jnp.int32) % N
    src = jnp.concatenate([edge_index[0], pad_src])
    dst = jnp.concatenate([edge_index[1], pad_dst])
    src_m = src.reshape(NCHUNKS, CH)
    dst_m = dst.reshape(NCHUNKS, CH)
    dst_d = dst[:32 * NCH * CH].reshape(32, NCH, CH)

    ones16 = jnp.ones((CH, 16), f32)
    zdeg = jnp.zeros((RT, 16), f32)
    zacc = jnp.zeros((RT, D), f32)

    degacc = _deg_kernel(dst_d, ones16, zdeg)
    h1a, h1b, h1c = _tc1(xp, W1f, degacc)
    s1 = _scatter_kernel(h1a, h1b, h1c, src_m, dst_m, zacc)
    h2a, h2b, h2c = _tc2(s1, h1a, h1b, h1c, degacc, *b1s, W2_0, W2_1, W2_2)
    s2 = _scatter_kernel(h2a, h2b, h2c, src_m, dst_m, zacc)
    out = _tc3(s2, h2a, h2b, h2c, degacc, *b2s, lin_W, lin_b.reshape(1, D))
    return out[:N]


# speedup vs baseline: 4.4678x; 1.0418x over previous
import functools

import jax
import jax.numpy as jnp
from jax import lax
from jax.experimental import pallas as pl
from jax.experimental.pallas import tpu as pltpu
from jax.experimental.pallas import tpu_sc as plsc

N = 10000
E = 320000
D = 128
F = 384
NP = 10240
RT = NP // 16
CH = 128
NCH = 79
NFAST = 80
NSLOW = 80
NCHUNKS = 16 * (NFAST + NSLOW)
EP = NCHUNKS * CH
BR = 512

_MESH = plsc.VectorSubcoreMesh(core_axis_name="c", subcore_axis_name="s")


@functools.partial(
    pl.kernel,
    mesh=_MESH,
    out_type=jax.ShapeDtypeStruct((2, NP, 16), jnp.float32),
    scratch_types=[
        pltpu.VMEM((NCH, CH), jnp.int32),
        pltpu.VMEM((CH, 16), jnp.float32),
        pltpu.VMEM_SHARED((NP, 16), jnp.float32),
    ],
    compiler_params=pltpu.CompilerParams(use_tc_tiling_on_sc=False),
)
def _deg_kernel(dst_hbm, ones_hbm, zeros_hbm, out_hbm, dst_v, ones_v, acc):
    c = lax.axis_index("c")
    s = lax.axis_index("s")
    wid = c * 16 + s
    pltpu.sync_copy(zeros_hbm, acc.at[pl.ds(s * RT, RT)])
    pltpu.sync_copy(dst_hbm.at[wid], dst_v)
    pltpu.sync_copy(ones_hbm, ones_v)
    plsc.subcore_barrier()

    def body(j, carry):
        pltpu.sync_copy(ones_v, acc.at[dst_v.at[j]], add=True)
        return carry

    lax.fori_loop(0, NCH, body, 0)
    plsc.subcore_barrier()
    pltpu.sync_copy(acc.at[pl.ds(s * RT, RT)], out_hbm.at[c, pl.ds(s * RT, RT)])


@functools.partial(
    pl.kernel,
    mesh=_MESH,
    out_type=jax.ShapeDtypeStruct((3, 2, NP, D), jnp.float32),
    scratch_types=[
        pltpu.VMEM((2, CH), jnp.int32),
        pltpu.VMEM((NFAST, CH), jnp.int32),
        pltpu.VMEM((CH, D), jnp.float32),
        pltpu.VMEM((CH, D), jnp.float32),
        pltpu.SemaphoreType.DMA,
        pltpu.SemaphoreType.DMA,
        pltpu.SemaphoreType.DMA,
        pltpu.SemaphoreType.DMA,
        pltpu.SemaphoreType.DMA,
        pltpu.SemaphoreType.DMA,
        pltpu.VMEM_SHARED((NP, D), jnp.float32),
    ],
)
def _scatter_kernel(tab0, tab1, tab2, src_hbm, dst_hbm, zeros_hbm, out_hbm,
                    sidx, dst_v, bufa, bufb, semg0, semg1, semi0, semi1,
                    semsc0, semsc1, acc):
    c = lax.axis_index("c")
    s = lax.axis_index("s")
    bufs = (bufa, bufb)
    semg = (semg0, semg1)
    semi = (semi0, semi1)
    semsc = (semsc0, semsc1)

    def run(base, n):
        pltpu.sync_copy(dst_hbm.at[pl.ds(base, n)], dst_v.at[pl.ds(0, n)])

        def start_sidx(j, b):
            pltpu.async_copy(src_hbm.at[base + j], sidx.at[b], semi[b])

        def wait_sidx(j, b):
            pltpu.make_async_copy(src_hbm.at[base + j], sidx.at[b],
                                  semi[b]).wait()

        for t, tab in enumerate((tab0, tab1, tab2)):
            pltpu.sync_copy(zeros_hbm, acc.at[pl.ds(s * RT, RT)])
            plsc.subcore_barrier()

            def start_gather(j, b, tab=tab):
                pltpu.async_copy(tab.at[sidx.at[b]], bufs[b], semg[b])

            def wait_gather(b, tab=tab):
                pltpu.make_async_copy(tab.at[sidx.at[b]], bufs[b],
                                      semg[b]).wait()

            def start_scatter(j, b):
                pltpu.async_copy(bufs[b], acc.at[dst_v.at[j]], semsc[b],
                                 add=True)

            def wait_scatter(j, b):
                pltpu.make_async_copy(bufs[b], acc.at[dst_v.at[j]],
                                      semsc[b]).wait()

            start_sidx(0, 0)
            start_sidx(1, 1)
            wait_sidx(0, 0)
            start_gather(0, 0)

            def step(j, b, guard2, guard1, first=False):
                def advance():
                    wait_sidx(j + 1, 1 - b)
                    if first:
                        @pl.when(j > 0)
                        def _():
                            wait_scatter(j - 1, 1 - b)
                    else:
                        wait_scatter(j - 1, 1 - b)
                    start_gather(j + 1, 1 - b)

                if guard1:
                    @pl.when(j + 1 < n)
                    def _():
                        advance()
                else:
                    advance()
                wait_gather(b)
                start_scatter(j, b)
                if guard2:
                    @pl.when(j + 2 < n)
                    def _():
                        start_sidx(j + 2, b)
                else:
                    start_sidx(j + 2, b)

            def body(g, carry):
                step(2 * g, 0, guard2=True, guard1=False, first=True)
                step(2 * g + 1, 1, guard2=True, guard1=True)
                return carry

            lax.fori_loop(0, n // 2, body, 0)
            wait_scatter(n - 2, 0)
            wait_scatter(n - 1, 1)

            plsc.subcore_barrier()
            pltpu.sync_copy(acc.at[pl.ds(s * RT, RT)],
                            out_hbm.at[t, c, pl.ds(s * RT, RT)])

    run((c * 16 + s) * NFAST, NFAST)


def _dinv_block(d_ref):
    deg = d_ref[0][:, 0:1] + d_ref[1][:, 0:1] + 1.0
    return lax.rsqrt(deg), jnp.sqrt(deg)


def _tc1_body(x_ref, w_ref, d_ref, o0_ref, o1_ref, o2_ref):
    dinv, _ = _dinv_block(d_ref)
    hp = jnp.dot(x_ref[...], w_ref[...],
                 preferred_element_type=jnp.float32) * dinv
    o0_ref[...] = hp[:, :D]
    o1_ref[...] = hp[:, D:2 * D]
    o2_ref[...] = hp[:, 2 * D:]


def _combine(s_ref, h_refs, d_ref, b_refs):
    dinv, sdeg = _dinv_block(d_ref)
    outs = []
    for t, (h_ref, b_ref) in enumerate(zip(h_refs, b_refs)):
        hp = h_ref[...]
        p = dinv * (s_ref[t, 0] + s_ref[t, 1] + hp)
        if t == 0:
            outs.append(p + b_ref[...])
        elif t == 1:
            outs.append(sdeg * hp - p + b_ref[...])
        else:
            outs.append(sdeg * hp + p + b_ref[...])
    return outs, dinv


def _tc2_body(s_ref, h0_ref, h1_ref, h2_ref, d_ref, b0_ref, b1_ref, b2_ref,
              w0_ref, w1_ref, w2_ref, o0_ref, o1_ref, o2_ref):
    cs, dinv = _combine(s_ref, (h0_ref, h1_ref, h2_ref), d_ref,
                        (b0_ref, b1_ref, b2_ref))
    for cmb, w_ref, o_ref in zip(cs, (w0_ref, w1_ref, w2_ref),
                                 (o0_ref, o1_ref, o2_ref)):
        o_ref[...] = jnp.dot(jnp.maximum(cmb, 0.0), w_ref[...],
                             preferred_element_type=jnp.float32) * dinv


def _tc3_body(s_ref, h0_ref, h1_ref, h2_ref, d_ref, b0_ref, b1_ref, b2_ref,
              w_ref, lb_ref, o_ref):
    cs, _ = _combine(s_ref, (h0_ref, h1_ref, h2_ref), d_ref,
                     (b0_ref, b1_ref, b2_ref))
    acc = lb_ref[...].astype(jnp.float32)
    for t, cmb in enumerate(cs):
        acc = acc + jnp.dot(cmb, w_ref[pl.ds(t * D, D), :],
                            preferred_element_type=jnp.float32)
    o_ref[...] = acc


_spec_rows = pl.BlockSpec((BR, D), lambda i: (i, 0))
_spec_full = lambda a, b: pl.BlockSpec((a, b), lambda i: (0, 0))
_spec_deg = pl.BlockSpec((2, BR, 16), lambda i: (0, i, 0))
_spec_s = pl.BlockSpec((3, 2, BR, D), lambda i: (0, 0, i, 0))
_out_rows = jax.ShapeDtypeStruct((NP, D), jnp.float32)

_tc1 = pl.pallas_call(
    _tc1_body,
    grid=(NP // BR,),
    in_specs=[_spec_rows, _spec_full(D, F), _spec_deg],
    out_specs=[_spec_rows] * 3,
    out_shape=[_out_rows] * 3,
)

_tc2 = pl.pallas_call(
    _tc2_body,
    grid=(NP // BR,),
    in_specs=[_spec_s, _spec_rows, _spec_rows, _spec_rows, _spec_deg]
             + [_spec_full(1, D)] * 3 + [_spec_full(D, D)] * 3,
    out_specs=[_spec_rows] * 3,
    out_shape=[_out_rows] * 3,
)

_tc3 = pl.pallas_call(
    _tc3_body,
    grid=(NP // BR,),
    in_specs=[_spec_s, _spec_rows, _spec_rows, _spec_rows, _spec_deg]
             + [_spec_full(1, D)] * 3 + [_spec_full(F, D), _spec_full(1, D)],
    out_specs=_spec_rows,
    out_shape=_out_rows,
)


def kernel(x, edge_index, W1_0, b1_0, W2_0, b2_0, W1_1, b1_1, W2_1, b2_1,
           W1_2, b1_2, W2_2, b2_2, lin_W, lin_b):
    f32 = jnp.float32
    W1f = jnp.concatenate([W1_0, W1_1, W1_2], axis=1)
    b1s = [b.reshape(1, D) for b in (b1_0, b1_1, b1_2)]
    b2s = [b.reshape(1, D) for b in (b2_0, b2_1, b2_2)]

    xp = jnp.pad(x, ((0, NP - N), (0, 0)))
    pad_dst = N + (jnp.arange(EP - E, dtype=jnp.int32) % (NP - N))
    pad_src = jnp.arange(EP - E, dtype=jnp.int32) % N
    src = jnp.concatenate([edge_index[0], pad_src])
    dst = jnp.concatenate([edge_index[1], pad_dst])
    src_m = src.reshape(NCHUNKS, CH)
    dst_m = dst.reshape(NCHUNKS, CH)
    dst_d = dst[:32 * NCH * CH].reshape(32, NCH, CH)

    ones16 = jnp.ones((CH, 16), f32)
    zdeg = jnp.zeros((RT, 16), f32)
    zacc = jnp.zeros((RT, D), f32)

    degacc = _deg_kernel(dst_d, ones16, zdeg)
    h1a, h1b, h1c = _tc1(xp, W1f, degacc)
    s1 = _scatter_kernel(h1a, h1b, h1c, src_m, dst_m, zacc)
    h2a, h2b, h2c = _tc2(s1, h1a, h1b, h1c, degacc, *b1s, W2_0, W2_1, W2_2)
    s2 = _scatter_kernel(h2a, h2b, h2c, src_m, dst_m, zacc)
    out = _tc3(s2, h2a, h2b, h2c, degacc, *b2s, lin_W, lin_b.reshape(1, D))
    return out[:N]
